# baseline (speedup 1.0000x reference)
import functools
import math

import jax
import jax.numpy as jnp
from jax import lax
from jax.experimental import pallas as pl
from jax.experimental.pallas import tpu as pltpu
from jax.experimental.pallas import tpu_sc as plsc

LANES = 16
NUM_CORES = 2
NUM_SUBCORES = 16
NUM_WORKERS = NUM_CORES * NUM_SUBCORES

SUCCESS_REWARD = 1.0
FAILURE_REWARD = 0.01
SHAPING_COEF = 0.5
LOG_SUCCESS = math.log(SUCCESS_REWARD)
LOG_FAILURE = math.log(FAILURE_REWARD)


def _pick_block(chunk: int) -> int:
    for cand in (8192, 8000, 6400, 5120, 4096, 4000, 3200, 2048, 1600,
                 1024, 800, 512, 400, 256, 128, 64, 32, 16):
        if chunk % cand == 0:
            return cand
    raise ValueError(f"no block size divides chunk={chunk}")


GROUP = 64


def _sc_segment_body(chunk, block, num_graphs,
                     sel_hbm, lab_hbm, seg_hbm,
                     tp_out, ps_out, ts_out,
                     sel_buf, lab_buf, seg_buf,
                     acc_tp, acc_ps, acc_ts, cur_ref):
    wid = lax.axis_index("s") * NUM_CORES + lax.axis_index("c")
    lane = lax.iota(jnp.int32, LANES)
    l0 = lane == 0
    l15 = lane == LANES - 1
    zeros = jnp.zeros((LANES,), jnp.float32)

    def zero_body(i, carry):
        acc_tp[pl.ds(i * LANES, LANES)] = zeros
        acc_ps[pl.ds(i * LANES, LANES)] = zeros
        acc_ts[pl.ds(i * LANES, LANES)] = zeros
        return carry

    lax.fori_loop(0, num_graphs // LANES, zero_body, 0)
    cur_ref[0] = 0

    def load_vals(o):
        pred = sel_buf[pl.ds(o, LANES)]
        lab = lab_buf[pl.ds(o, LANES)]
        tm = lab > 0.5
        targ = jnp.where(tm, 1.0, 0.0).astype(jnp.float32)
        pt = jnp.where(tm, pred, 0.0).astype(jnp.float32)
        return pt, pred, targ

    def general_vec(o):
        s = seg_buf[pl.ds(o + 8, LANES)]
        sprev = seg_buf[pl.ds(o + 7, LANES)]
        snext = seg_buf[pl.ds(o + 9, LANES)]
        m_end = (s != snext) | l15
        m_start = (s != sprev) | l0
        vals = load_vals(o)
        for acc, v in zip((acc_tp, acc_ps, acc_ts), vals):
            c = plsc.cumsum(v)
            plsc.addupdate_scatter(acc, [s], c, mask=m_end)
            plsc.addupdate_scatter(acc, [s], v - c, mask=m_start)

    def flush(a3):
        idx = jnp.full((LANES,), cur_ref[0], dtype=jnp.int32)
        for acc, av in zip((acc_tp, acc_ps, acc_ts), a3):
            tot = jnp.sum(av)
            totv = jnp.full((LANES,), tot, dtype=jnp.float32)
            plsc.addupdate_scatter(acc, [idx], totv, mask=l0)

    def group_body(g, a3):
        o = g * GROUP
        sv0 = seg_buf[pl.ds(o + 8, LANES)]
        svl = seg_buf[pl.ds(o + 8 + GROUP - LANES, LANES)]
        uniform = jnp.all((sv0 == svl) & (sv0 == cur_ref[0]))

        vals = [load_vals(o + k * LANES) for k in range(GROUP // LANES)]
        sums = [(vals[0][a] + vals[1][a]) + (vals[2][a] + vals[3][a])
                for a in range(3)]

        @pl.when(jnp.logical_not(uniform))
        def _():
            flush(a3)
            for k in range(GROUP // LANES):
                general_vec(o + k * LANES)
            cur_ref[0] = svl[LANES - 1]

        return tuple(
            jnp.where(uniform, av + sv, 0.0)
            for av, sv in zip(a3, sums))

    def blk_body(b, a3):
        base = wid * chunk + b * block
        pltpu.sync_copy(sel_hbm.at[pl.ds(base, block)], sel_buf)
        pltpu.sync_copy(lab_hbm.at[pl.ds(base, block)], lab_buf)
        pltpu.sync_copy(seg_hbm.at[pl.ds(base, block)],
                        seg_buf.at[pl.ds(8, block)])
        return lax.fori_loop(0, block // GROUP, group_body, a3)

    a3 = lax.fori_loop(0, chunk // block, blk_body, (zeros, zeros, zeros))
    flush(a3)

    pltpu.sync_copy(acc_tp, tp_out.at[wid])
    pltpu.sync_copy(acc_ps, ps_out.at[wid])
    pltpu.sync_copy(acc_ts, ts_out.at[wid])


def _segment_partials(selected_mask, edge_labels, edge_batch, num_graphs):
    num_edges = selected_mask.shape[0]
    assert num_edges % (NUM_WORKERS * LANES) == 0
    chunk = num_edges // NUM_WORKERS
    block = _pick_block(chunk)
    assert block % GROUP == 0
    mesh = plsc.VectorSubcoreMesh(core_axis_name="c", subcore_axis_name="s",
                                  num_cores=NUM_CORES,
                                  num_subcores=NUM_SUBCORES)
    acc = jax.ShapeDtypeStruct((NUM_WORKERS, num_graphs), jnp.float32)
    run = pl.kernel(
        functools.partial(_sc_segment_body, chunk, block, num_graphs),
        out_type=(acc, acc, acc),
        mesh=mesh,
        compiler_params=pltpu.CompilerParams(needs_layout_passes=False),
        scratch_types=(
            pltpu.VMEM((block,), jnp.float32),
            pltpu.VMEM((block,), jnp.float32),
            pltpu.VMEM((block + 16,), jnp.int32),
            pltpu.VMEM((num_graphs,), jnp.float32),
            pltpu.VMEM((num_graphs,), jnp.float32),
            pltpu.VMEM((num_graphs,), jnp.float32),
            pltpu.SMEM((1,), jnp.int32),
        ),
    )
    return run(selected_mask, edge_labels, edge_batch)


def _finalize_body(tp_ref, ps_ref, ts_ref, hit_ref,
                   reward_ref, logr_ref, succ_ref,
                   prec_ref, rec_ref, f1_ref):
    tp = jnp.sum(tp_ref[...], axis=0, keepdims=True)
    ps = jnp.sum(ps_ref[...], axis=0, keepdims=True)
    ts = jnp.sum(ts_ref[...], axis=0, keepdims=True)
    zeros = jnp.zeros_like(tp)
    prec = jnp.where(ps > 0, tp / jnp.maximum(ps, 1.0), zeros)
    rec = jnp.where(ts > 0, tp / jnp.maximum(ts, 1.0), zeros)
    f1 = 2.0 * prec * rec / (prec + rec + 1e-08)
    hit = hit_ref[...]
    logr = jnp.where(hit.astype(jnp.bool_),
                     jnp.float32(LOG_SUCCESS),
                     jnp.float32(LOG_FAILURE)) + SHAPING_COEF * f1
    reward_ref[...] = jnp.exp(logr)
    logr_ref[...] = logr
    succ_ref[...] = hit.astype(jnp.float32)
    prec_ref[...] = prec
    rec_ref[...] = rec
    f1_ref[...] = f1


def _finalize(tp_p, ps_p, ts_p, hit2d):
    g = hit2d.shape[1]
    out = jax.ShapeDtypeStruct((1, g), jnp.float32)
    return pl.pallas_call(
        _finalize_body,
        out_shape=(out,) * 6,
    )(tp_p, ps_p, ts_p, hit2d)


def kernel(selected_mask, edge_labels, edge_batch, answer_hit):
    num_graphs = answer_hit.shape[0]
    tp_p, ps_p, ts_p = _segment_partials(
        selected_mask.astype(jnp.float32),
        edge_labels.astype(jnp.float32),
        edge_batch.astype(jnp.int32),
        num_graphs,
    )
    outs = _finalize(tp_p, ps_p, ts_p,
                     answer_hit.astype(jnp.int32).reshape(1, num_graphs))
    return tuple(o.reshape(num_graphs) for o in outs)


# speedup vs baseline: 1.7691x; 1.7691x over previous
import functools
import math

import jax
import jax.numpy as jnp
from jax import lax
from jax.experimental import pallas as pl
from jax.experimental.pallas import tpu as pltpu
from jax.experimental.pallas import tpu_sc as plsc

LANES = 16
NUM_CORES = 2
NUM_SUBCORES = 16
NUM_WORKERS = NUM_CORES * NUM_SUBCORES

SUCCESS_REWARD = 1.0
FAILURE_REWARD = 0.01
SHAPING_COEF = 0.5
LOG_SUCCESS = math.log(SUCCESS_REWARD)
LOG_FAILURE = math.log(FAILURE_REWARD)


def _pick_block(chunk: int) -> int:
    for cand in (8192, 8000, 6400, 5120, 4096, 4000, 3200, 2048, 1600,
                 1024, 800, 512, 400, 256, 128, 64, 32, 16):
        if chunk % cand == 0:
            return cand
    raise ValueError(f"no block size divides chunk={chunk}")


GROUP = 64


def _sc_segment_body(chunk, block, num_graphs,
                     sel_hbm, lab_hbm, seg_hbm,
                     tp_out, ps_out, ts_out,
                     sel_a, lab_a, seg_a, sel_b, lab_b, seg_b,
                     acc_tp, acc_ps, acc_ts, cur_ref, sem_a, sem_b):
    wid = lax.axis_index("s") * NUM_CORES + lax.axis_index("c")
    lane = lax.iota(jnp.int32, LANES)
    l0 = lane == 0
    l15 = lane == LANES - 1
    zeros = jnp.zeros((LANES,), jnp.float32)

    def zero_body(i, carry):
        acc_tp[pl.ds(i * LANES, LANES)] = zeros
        acc_ps[pl.ds(i * LANES, LANES)] = zeros
        acc_ts[pl.ds(i * LANES, LANES)] = zeros
        return carry

    lax.fori_loop(0, num_graphs // LANES, zero_body, 0)
    cur_ref[0] = 0

    def load_vals(bufs, o):
        sel_buf, lab_buf, _ = bufs
        pred = sel_buf[pl.ds(o, LANES)]
        lab = lab_buf[pl.ds(o, LANES)]
        tm = lab > 0.5
        targ = jnp.where(tm, 1.0, 0.0).astype(jnp.float32)
        pt = jnp.where(tm, pred, 0.0).astype(jnp.float32)
        return pt, pred, targ

    def general_vec(bufs, o):
        seg_buf = bufs[2]
        s = seg_buf[pl.ds(o + 8, LANES)]
        sprev = seg_buf[pl.ds(o + 7, LANES)]
        snext = seg_buf[pl.ds(o + 9, LANES)]
        m_end = (s != snext) | l15
        m_start = (s != sprev) | l0
        vals = load_vals(bufs, o)
        for acc, v in zip((acc_tp, acc_ps, acc_ts), vals):
            c = plsc.cumsum(v)
            plsc.addupdate_scatter(acc, [s], c, mask=m_end)
            plsc.addupdate_scatter(acc, [s], v - c, mask=m_start)

    def flush(a3):
        idx = jnp.full((LANES,), cur_ref[0], dtype=jnp.int32)
        for acc, av in zip((acc_tp, acc_ps, acc_ts), a3):
            tot = jnp.sum(av)
            totv = jnp.full((LANES,), tot, dtype=jnp.float32)
            plsc.addupdate_scatter(acc, [idx], totv, mask=l0)

    def make_group_body(bufs):
        def group_body(g, a3):
            o = g * GROUP
            seg_buf = bufs[2]
            s_first = seg_buf[pl.ds(o + 8, LANES)][0]
            s_last = seg_buf[pl.ds(o + 8 + GROUP - LANES, LANES)][LANES - 1]
            uniform = (s_first == cur_ref[0]) & (s_last == s_first)

            vals = [load_vals(bufs, o + k * LANES)
                    for k in range(GROUP // LANES)]
            sums = [(vals[0][a] + vals[1][a]) + (vals[2][a] + vals[3][a])
                    for a in range(3)]

            @pl.when(jnp.logical_not(uniform))
            def _():
                flush(a3)
                for k in range(GROUP // LANES):
                    general_vec(bufs, o + k * LANES)
                cur_ref[0] = s_last

            return tuple(
                jnp.where(uniform, av + sv, 0.0)
                for av, sv in zip(a3, sums))

        return group_body

    bufs_a = (sel_a, lab_a, seg_a)
    bufs_b = (sel_b, lab_b, seg_b)
    body_a = make_group_body(bufs_a)
    body_b = make_group_body(bufs_b)

    def issue(bufs, sem, b):
        base = wid * chunk + b * block
        pltpu.async_copy(sel_hbm.at[pl.ds(base, block)], bufs[0], sem)
        pltpu.async_copy(lab_hbm.at[pl.ds(base, block)], bufs[1], sem)
        pltpu.async_copy(seg_hbm.at[pl.ds(base, block)],
                         bufs[2].at[pl.ds(8, block)], sem)

    def drain(bufs, sem):
        pltpu.make_async_copy(sel_hbm.at[pl.ds(0, block)], bufs[0], sem).wait()
        pltpu.make_async_copy(lab_hbm.at[pl.ds(0, block)], bufs[1], sem).wait()
        pltpu.make_async_copy(seg_hbm.at[pl.ds(0, block)],
                              bufs[2].at[pl.ds(8, block)], sem).wait()

    nb = chunk // block
    n_groups = block // GROUP
    issue(bufs_a, sem_a, 0)

    def pair_body(p, a3):
        b = 2 * p
        drain(bufs_a, sem_a)
        issue(bufs_b, sem_b, b + 1)
        a3 = lax.fori_loop(0, n_groups, body_a, a3)
        drain(bufs_b, sem_b)

        @pl.when(b + 2 < nb)
        def _():
            issue(bufs_a, sem_a, b + 2)

        return lax.fori_loop(0, n_groups, body_b, a3)

    a3 = lax.fori_loop(0, nb // 2, pair_body, (zeros, zeros, zeros))
    if nb % 2 == 1:
        drain(bufs_a, sem_a)
        a3 = lax.fori_loop(0, n_groups, body_a, a3)
    flush(a3)

    pltpu.sync_copy(acc_tp, tp_out.at[wid])
    pltpu.sync_copy(acc_ps, ps_out.at[wid])
    pltpu.sync_copy(acc_ts, ts_out.at[wid])


def _segment_partials(selected_mask, edge_labels, edge_batch, num_graphs):
    num_edges = selected_mask.shape[0]
    assert num_edges % (NUM_WORKERS * LANES) == 0
    chunk = num_edges // NUM_WORKERS
    block = _pick_block(chunk)
    assert block % GROUP == 0
    mesh = plsc.VectorSubcoreMesh(core_axis_name="c", subcore_axis_name="s",
                                  num_cores=NUM_CORES,
                                  num_subcores=NUM_SUBCORES)
    acc = jax.ShapeDtypeStruct((NUM_WORKERS, num_graphs), jnp.float32)
    run = pl.kernel(
        functools.partial(_sc_segment_body, chunk, block, num_graphs),
        out_type=(acc, acc, acc),
        mesh=mesh,
        compiler_params=pltpu.CompilerParams(needs_layout_passes=False),
        scratch_types=(
            pltpu.VMEM((block,), jnp.float32),
            pltpu.VMEM((block,), jnp.float32),
            pltpu.VMEM((block + 16,), jnp.int32),
            pltpu.VMEM((block,), jnp.float32),
            pltpu.VMEM((block,), jnp.float32),
            pltpu.VMEM((block + 16,), jnp.int32),
            pltpu.VMEM((num_graphs,), jnp.float32),
            pltpu.VMEM((num_graphs,), jnp.float32),
            pltpu.VMEM((num_graphs,), jnp.float32),
            pltpu.SMEM((1,), jnp.int32),
            pltpu.SemaphoreType.DMA,
            pltpu.SemaphoreType.DMA,
        ),
    )
    return run(selected_mask, edge_labels, edge_batch)


def _finalize_body(tp_ref, ps_ref, ts_ref, hit_ref,
                   reward_ref, logr_ref, succ_ref,
                   prec_ref, rec_ref, f1_ref):
    tp = jnp.sum(tp_ref[...], axis=0, keepdims=True)
    ps = jnp.sum(ps_ref[...], axis=0, keepdims=True)
    ts = jnp.sum(ts_ref[...], axis=0, keepdims=True)
    zeros = jnp.zeros_like(tp)
    prec = jnp.where(ps > 0, tp / jnp.maximum(ps, 1.0), zeros)
    rec = jnp.where(ts > 0, tp / jnp.maximum(ts, 1.0), zeros)
    f1 = 2.0 * prec * rec / (prec + rec + 1e-08)
    hit = hit_ref[...]
    logr = jnp.where(hit.astype(jnp.bool_),
                     jnp.float32(LOG_SUCCESS),
                     jnp.float32(LOG_FAILURE)) + SHAPING_COEF * f1
    reward_ref[...] = jnp.exp(logr)
    logr_ref[...] = logr
    succ_ref[...] = hit.astype(jnp.float32)
    prec_ref[...] = prec
    rec_ref[...] = rec
    f1_ref[...] = f1


def _finalize(tp_p, ps_p, ts_p, hit2d):
    g = hit2d.shape[1]
    out = jax.ShapeDtypeStruct((1, g), jnp.float32)
    return pl.pallas_call(
        _finalize_body,
        out_shape=(out,) * 6,
    )(tp_p, ps_p, ts_p, hit2d)


def kernel(selected_mask, edge_labels, edge_batch, answer_hit):
    num_graphs = answer_hit.shape[0]
    tp_p, ps_p, ts_p = _segment_partials(
        selected_mask.astype(jnp.float32),
        edge_labels.astype(jnp.float32),
        edge_batch.astype(jnp.int32),
        num_graphs,
    )
    outs = _finalize(tp_p, ps_p, ts_p,
                     answer_hit.astype(jnp.int32).reshape(1, num_graphs))
    return tuple(o.reshape(num_graphs) for o in outs)


# speedup vs baseline: 2.2851x; 1.2917x over previous
import functools
import math

import jax
import jax.numpy as jnp
from jax import lax
from jax.experimental import pallas as pl
from jax.experimental.pallas import tpu as pltpu
from jax.experimental.pallas import tpu_sc as plsc

LANES = 16
NUM_CORES = 2
NUM_SUBCORES = 16
NUM_WORKERS = NUM_CORES * NUM_SUBCORES

SUCCESS_REWARD = 1.0
FAILURE_REWARD = 0.01
SHAPING_COEF = 0.5
LOG_SUCCESS = math.log(SUCCESS_REWARD)
LOG_FAILURE = math.log(FAILURE_REWARD)


def _pick_block(chunk: int) -> int:
    for cand in (8192, 8000, 6400, 5120, 4096, 4000, 3200, 2048, 1600,
                 1024, 800, 512, 400, 256, 128, 64, 32, 16):
        if chunk % cand == 0:
            return cand
    raise ValueError(f"no block size divides chunk={chunk}")


GROUP = 160


def _tree_sum(xs):
    xs = list(xs)
    while len(xs) > 1:
        nxt = [xs[i] + xs[i + 1] for i in range(0, len(xs) - 1, 2)]
        if len(xs) % 2 == 1:
            nxt.append(xs[-1])
        xs = nxt
    return xs[0]


def _sc_segment_body(chunk, block, num_graphs,
                     sel_hbm, lab_hbm, seg_hbm,
                     tp_out, ps_out, ts_out,
                     sel_a, lab_a, seg_a, sel_b, lab_b, seg_b,
                     acc_tp, acc_ps, acc_ts, cur_ref, sem_a, sem_b):
    wid = lax.axis_index("s") * NUM_CORES + lax.axis_index("c")
    lane = lax.iota(jnp.int32, LANES)
    l0 = lane == 0
    l15 = lane == LANES - 1
    zeros = jnp.zeros((LANES,), jnp.float32)

    def zero_body(i, carry):
        acc_tp[pl.ds(i * LANES, LANES)] = zeros
        acc_ps[pl.ds(i * LANES, LANES)] = zeros
        acc_ts[pl.ds(i * LANES, LANES)] = zeros
        return carry

    lax.fori_loop(0, num_graphs // LANES, zero_body, 0)
    cur_ref[0] = 0

    def load_vals(bufs, o):
        sel_buf, lab_buf, _ = bufs
        pred = sel_buf[pl.ds(o, LANES)]
        lab = lab_buf[pl.ds(o, LANES)]
        tm = lab > 0.5
        targ = jnp.where(tm, 1.0, 0.0).astype(jnp.float32)
        pt = jnp.where(tm, pred, 0.0).astype(jnp.float32)
        return pt, pred, targ

    def general_vec(bufs, o):
        seg_buf = bufs[2]
        s = seg_buf[pl.ds(o + 8, LANES)]
        sprev = seg_buf[pl.ds(o + 7, LANES)]
        snext = seg_buf[pl.ds(o + 9, LANES)]
        m_end = (s != snext) | l15
        m_start = (s != sprev) | l0
        vals = load_vals(bufs, o)
        for acc, v in zip((acc_tp, acc_ps, acc_ts), vals):
            c = plsc.cumsum(v)
            plsc.addupdate_scatter(acc, [s], c, mask=m_end)
            plsc.addupdate_scatter(acc, [s], v - c, mask=m_start)

    def flush(a3):
        idx = jnp.full((LANES,), cur_ref[0], dtype=jnp.int32)
        for acc, av in zip((acc_tp, acc_ps, acc_ts), a3):
            tot = jnp.sum(av)
            totv = jnp.full((LANES,), tot, dtype=jnp.float32)
            plsc.addupdate_scatter(acc, [idx], totv, mask=l0)

    def make_group_body(bufs):
        def group_body(g, a3):
            o = g * GROUP
            seg_buf = bufs[2]
            s_first = seg_buf[pl.ds(o + 8, LANES)][0]
            s_last = seg_buf[pl.ds(o + 8 + GROUP - LANES, LANES)][LANES - 1]
            uniform = (s_first == cur_ref[0]) & (s_last == s_first)

            vals = [load_vals(bufs, o + k * LANES)
                    for k in range(GROUP // LANES)]
            sums = [_tree_sum([v[a] for v in vals]) for a in range(3)]

            @pl.when(jnp.logical_not(uniform))
            def _():
                flush(a3)
                for k in range(GROUP // LANES):
                    general_vec(bufs, o + k * LANES)
                cur_ref[0] = s_last

            return tuple(
                jnp.where(uniform, av + sv, 0.0)
                for av, sv in zip(a3, sums))

        return group_body

    bufs_a = (sel_a, lab_a, seg_a)
    bufs_b = (sel_b, lab_b, seg_b)
    body_a = make_group_body(bufs_a)
    body_b = make_group_body(bufs_b)

    def issue(bufs, sem, b):
        base = wid * chunk + b * block
        pltpu.async_copy(sel_hbm.at[pl.ds(base, block)], bufs[0], sem)
        pltpu.async_copy(lab_hbm.at[pl.ds(base, block)], bufs[1], sem)
        pltpu.async_copy(seg_hbm.at[pl.ds(base, block)],
                         bufs[2].at[pl.ds(8, block)], sem)

    def drain(bufs, sem):
        pltpu.make_async_copy(sel_hbm.at[pl.ds(0, block)], bufs[0], sem).wait()
        pltpu.make_async_copy(lab_hbm.at[pl.ds(0, block)], bufs[1], sem).wait()
        pltpu.make_async_copy(seg_hbm.at[pl.ds(0, block)],
                              bufs[2].at[pl.ds(8, block)], sem).wait()

    nb = chunk // block
    n_groups = block // GROUP
    issue(bufs_a, sem_a, 0)

    def pair_body(p, a3):
        b = 2 * p
        drain(bufs_a, sem_a)
        issue(bufs_b, sem_b, b + 1)
        a3 = lax.fori_loop(0, n_groups, body_a, a3)
        drain(bufs_b, sem_b)

        @pl.when(b + 2 < nb)
        def _():
            issue(bufs_a, sem_a, b + 2)

        return lax.fori_loop(0, n_groups, body_b, a3)

    a3 = lax.fori_loop(0, nb // 2, pair_body, (zeros, zeros, zeros))
    if nb % 2 == 1:
        drain(bufs_a, sem_a)
        a3 = lax.fori_loop(0, n_groups, body_a, a3)
    flush(a3)

    pltpu.sync_copy(acc_tp, tp_out.at[wid])
    pltpu.sync_copy(acc_ps, ps_out.at[wid])
    pltpu.sync_copy(acc_ts, ts_out.at[wid])


def _segment_partials(selected_mask, edge_labels, edge_batch, num_graphs):
    num_edges = selected_mask.shape[0]
    assert num_edges % (NUM_WORKERS * LANES) == 0
    chunk = num_edges // NUM_WORKERS
    block = _pick_block(chunk)
    assert block % GROUP == 0
    mesh = plsc.VectorSubcoreMesh(core_axis_name="c", subcore_axis_name="s",
                                  num_cores=NUM_CORES,
                                  num_subcores=NUM_SUBCORES)
    acc = jax.ShapeDtypeStruct((NUM_WORKERS, num_graphs), jnp.float32)
    run = pl.kernel(
        functools.partial(_sc_segment_body, chunk, block, num_graphs),
        out_type=(acc, acc, acc),
        mesh=mesh,
        compiler_params=pltpu.CompilerParams(needs_layout_passes=False),
        scratch_types=(
            pltpu.VMEM((block,), jnp.float32),
            pltpu.VMEM((block,), jnp.float32),
            pltpu.VMEM((block + 16,), jnp.int32),
            pltpu.VMEM((block,), jnp.float32),
            pltpu.VMEM((block,), jnp.float32),
            pltpu.VMEM((block + 16,), jnp.int32),
            pltpu.VMEM((num_graphs,), jnp.float32),
            pltpu.VMEM((num_graphs,), jnp.float32),
            pltpu.VMEM((num_graphs,), jnp.float32),
            pltpu.SMEM((1,), jnp.int32),
            pltpu.SemaphoreType.DMA,
            pltpu.SemaphoreType.DMA,
        ),
    )
    return run(selected_mask, edge_labels, edge_batch)


def _finalize_body(tp_ref, ps_ref, ts_ref, hit_ref,
                   reward_ref, logr_ref, succ_ref,
                   prec_ref, rec_ref, f1_ref):
    tp = jnp.sum(tp_ref[...], axis=0, keepdims=True)
    ps = jnp.sum(ps_ref[...], axis=0, keepdims=True)
    ts = jnp.sum(ts_ref[...], axis=0, keepdims=True)
    zeros = jnp.zeros_like(tp)
    prec = jnp.where(ps > 0, tp / jnp.maximum(ps, 1.0), zeros)
    rec = jnp.where(ts > 0, tp / jnp.maximum(ts, 1.0), zeros)
    f1 = 2.0 * prec * rec / (prec + rec + 1e-08)
    hit = hit_ref[...]
    logr = jnp.where(hit.astype(jnp.bool_),
                     jnp.float32(LOG_SUCCESS),
                     jnp.float32(LOG_FAILURE)) + SHAPING_COEF * f1
    reward_ref[...] = jnp.exp(logr)
    logr_ref[...] = logr
    succ_ref[...] = hit.astype(jnp.float32)
    prec_ref[...] = prec
    rec_ref[...] = rec
    f1_ref[...] = f1


def _finalize(tp_p, ps_p, ts_p, hit2d):
    g = hit2d.shape[1]
    out = jax.ShapeDtypeStruct((1, g), jnp.float32)
    return pl.pallas_call(
        _finalize_body,
        out_shape=(out,) * 6,
    )(tp_p, ps_p, ts_p, hit2d)


def kernel(selected_mask, edge_labels, edge_batch, answer_hit):
    num_graphs = answer_hit.shape[0]
    tp_p, ps_p, ts_p = _segment_partials(
        selected_mask.astype(jnp.float32),
        edge_labels.astype(jnp.float32),
        edge_batch.astype(jnp.int32),
        num_graphs,
    )
    outs = _finalize(tp_p, ps_p, ts_p,
                     answer_hit.astype(jnp.int32).reshape(1, num_graphs))
    return tuple(o.reshape(num_graphs) for o in outs)
